# baseline (speedup 1.0000x reference)
import functools
import math

import jax
import jax.numpy as jnp
from jax import lax
from jax.experimental import pallas as pl
from jax.experimental.pallas import tpu as pltpu
from jax.experimental.pallas import tpu_sc as plsc

V = 100000
N = 1024
PAD = 0
SMOOTHING = 0.1
CONF = 1.0 - SMOOTHING
EPS = SMOOTHING / (V - 2)
CROW = SMOOTHING * math.log(EPS) + CONF * math.log(CONF)

_NC = 2
_NS = 16
NW = _NC * _NS
BPW = N // NW

SLAB = 512
NSLAB = 76
SCW = NSLAB * SLAB

VB = 512
GRID = -(-(V - SCW) // (2 * VB))
CUT = GRID * VB
BOFF = (CUT + SCW) // VB


def _sum128(x):
    s = x[:, 0:128]
    for g in range(1, x.shape[1] // 128):
        s = s + x[:, g * 128:(g + 1) * 128]
    return s


def _tc_body(tgt_ref, xa, xb, out_ref, acc_ref, s0_ref):
    j = pl.program_id(0)
    nlast = pl.num_programs(0) - 1

    @pl.when(j == 0)
    def _init():
        m = jnp.minimum(tgt_ref[...], 1).astype(jnp.float32)
        acc_ref[...] = jnp.zeros_like(acc_ref)
        s0_ref[0, 0] = jnp.sum(xa[:, 0:1] * m)

    s = _sum128(xa[...])

    @pl.when(j != nlast)
    def _main():
        acc_ref[...] += s + _sum128(xb[...])

    @pl.when(j == nlast)
    def _tail():
        x = xb[...]
        col = lax.broadcasted_iota(jnp.int32, x.shape, 1) + (j + BOFF) * VB
        acc_ref[...] += s + _sum128(jnp.where(col < V, x, 0.0))
        m = jnp.minimum(tgt_ref[...], 1).astype(jnp.float32)
        out_ref[0, 0] = EPS * s0_ref[0, 0] - EPS * jnp.sum(acc_ref[...] * m)


@functools.cache
def _make_sc_part():
    return functools.partial(
        pl.kernel,
        out_type=jax.ShapeDtypeStruct((NW, 16), jnp.float32),
        mesh=plsc.VectorSubcoreMesh(core_axis_name="c", subcore_axis_name="s"),
        scratch_types=[
            pltpu.VMEM((BPW,), jnp.int32),
            pltpu.VMEM((BPW, 8, 128), jnp.float32),
            pltpu.VMEM((8, SLAB), jnp.float32),
            pltpu.VMEM((8, SLAB), jnp.float32),
            pltpu.VMEM((16,), jnp.float32),
            pltpu.SemaphoreType.DMA,
            pltpu.SemaphoreType.DMA,
            pltpu.SemaphoreType.DMA,
        ],
    )(_sc_body)


def _sc_body(tgt_hbm, x_hbm, out_hbm, tgt_v, blk_v, buf_a, buf_b, acc_v,
             sem, sem_a, sem_b):
    wid = lax.axis_index("s") * _NC + lax.axis_index("c")
    base = wid * BPW
    pltpu.sync_copy(tgt_hbm.at[pl.ds(base, BPW)], tgt_v)
    t16s = [tgt_v[pl.ds(0, 16)], tgt_v[pl.ds(16, 16)]]

    copies = []
    for c in range(BPW // 16):
        t16 = t16s[c]
        for i in range(16):
            k = c * 16 + i
            ti = t16[i]
            nb = pl.multiple_of(base + (k & ~7), 8)
            cb = pl.multiple_of(jnp.bitwise_and(ti, jnp.int32(~127)), 128)
            cp = pltpu.make_async_copy(
                x_hbm.at[pl.ds(nb, 8), pl.ds(cb, 128)], blk_v.at[k], sem)
            cp.start()
            copies.append(cp)

    def _start(slab_idx, buf, s):
        col = pl.multiple_of(CUT + slab_idx * SLAB, 128)
        rb = pl.multiple_of(base + _start.tr * 8, 8)
        pltpu.make_async_copy(
            x_hbm.at[pl.ds(rb, 8), pl.ds(col, SLAB)], buf, s).start()

    def _wait(buf, s):
        pltpu.make_async_copy(
            x_hbm.at[pl.ds(0, 8), pl.ds(0, SLAB)], buf, s).wait()

    def _accum(buf, accs, nch):
        out = []
        for r in range(8):
            a = accs[r]
            for g in range(nch):
                a = a + buf[r, pl.ds(g * 16, 16)]
            out.append(a)
        return tuple(out)

    sctot = jnp.zeros((16,), jnp.float32)
    for tr in range(4):
        _start.tr = tr
        _start(0, buf_a, sem_a)
        _start(1, buf_b, sem_b)

        def _loop(i, accs):
            _wait(buf_a, sem_a)
            accs = _accum(buf_a, accs, SLAB // 16)

            @pl.when(2 * i + 2 < NSLAB)
            def _():
                _start(2 * i + 2, buf_a, sem_a)

            _wait(buf_b, sem_b)
            accs = _accum(buf_b, accs, SLAB // 16)

            @pl.when(2 * i + 3 < NSLAB)
            def _():
                _start(2 * i + 3, buf_b, sem_b)

            return accs

        accs = lax.fori_loop(
            0, NSLAB // 2, _loop,
            tuple(jnp.zeros((16,), jnp.float32) for _ in range(8)))

        t16 = t16s[tr // 2]
        for r in range(8):
            ti = t16[(tr % 2) * 8 + r]
            mf = jnp.minimum(ti, 1).astype(jnp.float32)
            sctot = sctot + mf * accs[r]

    for cp in copies:
        cp.wait()
    gacc = jnp.zeros((16,), jnp.float32)
    lanes = lax.iota(jnp.int32, 16)
    for c in range(BPW // 16):
        t16 = t16s[c]
        for i in range(16):
            k = c * 16 + i
            ti = t16[i]
            q = pl.multiple_of(jnp.bitwise_and(jnp.right_shift(ti, 4), 7) * 16, 16)
            chunk = blk_v[k, k & 7, pl.ds(q, 16)]
            oh = (1 - jnp.minimum(jnp.abs(lanes - jnp.bitwise_and(ti, 15)), 1)) * jnp.minimum(ti, 1)
            gacc = gacc + oh.astype(jnp.float32) * ((EPS - CONF) * chunk + CROW)
    acc_v[...] = gacc + (-EPS) * sctot
    pltpu.sync_copy(acc_v, out_hbm.at[wid])


def kernel(x, target):
    tgt = target.astype(jnp.int32)
    tc_out = pl.pallas_call(
        _tc_body,
        grid=(GRID,),
        in_specs=[
            pl.BlockSpec((N, 1), lambda j: (0, 0)),
            pl.BlockSpec((N, VB), lambda j: (0, j)),
            pl.BlockSpec((N, VB), lambda j: (0, j + BOFF)),
        ],
        out_specs=pl.BlockSpec((1, 1), lambda j: (0, 0), memory_space=pltpu.SMEM),
        out_shape=jax.ShapeDtypeStruct((1, 1), jnp.float32),
        scratch_shapes=[
            pltpu.VMEM((N, 128), jnp.float32),
            pltpu.SMEM((1, 1), jnp.float32),
        ],
    )(tgt.reshape(N, 1), x, x)
    sc_out = _make_sc_part()(tgt, x)
    return tc_out[0, 0] + jnp.sum(sc_out)


# speedup vs baseline: 1.0040x; 1.0040x over previous
import functools
import math

import jax
import jax.numpy as jnp
from jax import lax
from jax.experimental import pallas as pl
from jax.experimental.pallas import tpu as pltpu
from jax.experimental.pallas import tpu_sc as plsc

V = 100000
N = 1024
PAD = 0
SMOOTHING = 0.1
CONF = 1.0 - SMOOTHING
EPS = SMOOTHING / (V - 2)
CROW = SMOOTHING * math.log(EPS) + CONF * math.log(CONF)

_NC = 2
_NS = 16
NW = _NC * _NS
BPW = N // NW

SLAB = 512
NSLAB = 76
SCW = NSLAB * SLAB

VB = 512
GRID = -(-(V - SCW) // (2 * VB))
CUT = GRID * VB
BOFF = (CUT + SCW) // VB


def _sum128(x):
    s = x[:, 0:128]
    for g in range(1, x.shape[1] // 128):
        s = s + x[:, g * 128:(g + 1) * 128]
    return s


def _tc_body(tgt_ref, xa, xb, out_ref, acc_ref, s0_ref):
    j = pl.program_id(0)
    nlast = pl.num_programs(0) - 1

    @pl.when(j == 0)
    def _init():
        m = jnp.minimum(tgt_ref[...], 1).astype(jnp.float32)
        acc_ref[...] = jnp.zeros_like(acc_ref)
        s0_ref[0, 0] = jnp.sum(xa[:, 0:1] * m)

    s = _sum128(xa[...])

    @pl.when(j != nlast)
    def _main():
        acc_ref[...] += s + _sum128(xb[...])

    @pl.when(j == nlast)
    def _tail():
        x = xb[...]
        col = lax.broadcasted_iota(jnp.int32, x.shape, 1) + (j + BOFF) * VB
        acc_ref[...] += s + _sum128(jnp.where(col < V, x, 0.0))
        m = jnp.minimum(tgt_ref[...], 1).astype(jnp.float32)
        out_ref[0, 0] = EPS * s0_ref[0, 0] - EPS * jnp.sum(acc_ref[...] * m)


@functools.cache
def _make_sc_part():
    return functools.partial(
        pl.kernel,
        out_type=jax.ShapeDtypeStruct((NW, 16), jnp.float32),
        mesh=plsc.VectorSubcoreMesh(core_axis_name="c", subcore_axis_name="s"),
        scratch_types=[
            pltpu.VMEM((BPW,), jnp.int32),
            pltpu.VMEM((BPW, 8, 128), jnp.float32),
            pltpu.VMEM((8, SLAB), jnp.float32),
            pltpu.VMEM((8, SLAB), jnp.float32),
            pltpu.VMEM((16,), jnp.float32),
            pltpu.SemaphoreType.DMA,
            pltpu.SemaphoreType.DMA,
            pltpu.SemaphoreType.DMA,
        ],
    )(_sc_body)


def _sc_body(tgt_hbm, x_hbm, out_hbm, tgt_v, blk_v, buf_a, buf_b, acc_v,
             sem, sem_a, sem_b):
    wid = lax.axis_index("s") * _NC + lax.axis_index("c")
    base = wid * BPW
    pltpu.sync_copy(tgt_hbm.at[pl.ds(base, BPW)], tgt_v)
    t16s = [tgt_v[pl.ds(0, 16)], tgt_v[pl.ds(16, 16)]]

    copies = []
    for c in range(BPW // 16):
        t16 = t16s[c]
        for i in range(16):
            k = c * 16 + i
            ti = t16[i]
            nb = pl.multiple_of(base + (k & ~7), 8)
            cb = pl.multiple_of(jnp.bitwise_and(ti, jnp.int32(~127)), 128)
            cp = pltpu.make_async_copy(
                x_hbm.at[pl.ds(nb, 8), pl.ds(cb, 128)], blk_v.at[k], sem)
            cp.start()
            copies.append(cp)

    def _start(slab_idx, buf, s):
        col = pl.multiple_of(CUT + slab_idx * SLAB, 128)
        rb = pl.multiple_of(base + _start.tr * 8, 8)
        pltpu.make_async_copy(
            x_hbm.at[pl.ds(rb, 8), pl.ds(col, SLAB)], buf, s).start()

    def _wait(buf, s):
        pltpu.make_async_copy(
            x_hbm.at[pl.ds(0, 8), pl.ds(0, SLAB)], buf, s).wait()

    def _accum(buf, accs, nch):
        out = []
        for r in range(8):
            a = accs[r]
            for g in range(nch):
                a = a + buf[r, pl.ds(g * 16, 16)]
            out.append(a)
        return tuple(out)

    sctot = jnp.zeros((16,), jnp.float32)
    for tr in range(4):
        _start.tr = tr
        _start(0, buf_a, sem_a)
        _start(1, buf_b, sem_b)

        def _loop(i, accs):
            _wait(buf_a, sem_a)
            accs = _accum(buf_a, accs, SLAB // 16)

            @pl.when(2 * i + 2 < NSLAB)
            def _():
                _start(2 * i + 2, buf_a, sem_a)

            _wait(buf_b, sem_b)
            accs = _accum(buf_b, accs, SLAB // 16)

            @pl.when(2 * i + 3 < NSLAB)
            def _():
                _start(2 * i + 3, buf_b, sem_b)

            return accs

        accs = lax.fori_loop(
            0, NSLAB // 2, _loop,
            tuple(jnp.zeros((16,), jnp.float32) for _ in range(8)))

        t16 = t16s[tr // 2]
        for r in range(8):
            ti = t16[(tr % 2) * 8 + r]
            mf = jnp.minimum(ti, 1).astype(jnp.float32)
            sctot = sctot + mf * accs[r]

    for cp in copies:
        cp.wait()
    gacc = jnp.zeros((16,), jnp.float32)
    lanes = lax.iota(jnp.int32, 16)
    for c in range(BPW // 16):
        t16 = t16s[c]
        for i in range(16):
            k = c * 16 + i
            ti = t16[i]
            q = pl.multiple_of(jnp.bitwise_and(jnp.right_shift(ti, 4), 7) * 16, 16)
            chunk = blk_v[k, k & 7, pl.ds(q, 16)]
            oh = (1 - jnp.minimum(jnp.abs(lanes - jnp.bitwise_and(ti, 15)), 1)) * jnp.minimum(ti, 1)
            gacc = gacc + oh.astype(jnp.float32) * ((EPS - CONF) * chunk + CROW)
    acc_v[...] = gacc + (-EPS) * sctot
    pltpu.sync_copy(acc_v, out_hbm.at[wid])


def kernel(x, target):
    tgt = target.astype(jnp.int32)
    sc_out = _make_sc_part()(tgt, x)
    tc_out = pl.pallas_call(
        _tc_body,
        grid=(GRID,),
        in_specs=[
            pl.BlockSpec((N, 1), lambda j: (0, 0)),
            pl.BlockSpec((N, VB), lambda j: (0, j)),
            pl.BlockSpec((N, VB), lambda j: (0, j + BOFF)),
        ],
        out_specs=pl.BlockSpec((1, 1), lambda j: (0, 0), memory_space=pltpu.SMEM),
        out_shape=jax.ShapeDtypeStruct((1, 1), jnp.float32),
        scratch_shapes=[
            pltpu.VMEM((N, 128), jnp.float32),
            pltpu.SMEM((1, 1), jnp.float32),
        ],
    )(tgt.reshape(N, 1), x, x)
    return tc_out[0, 0] + jnp.sum(sc_out)


# speedup vs baseline: 2.8400x; 2.8287x over previous
import functools
import math

import jax
import jax.numpy as jnp
from jax import lax
from jax.experimental import pallas as pl
from jax.experimental.pallas import tpu as pltpu
from jax.experimental.pallas import tpu_sc as plsc

V = 100000
N = 1024
PAD = 0
SMOOTHING = 0.1
CONF = 1.0 - SMOOTHING
EPS = SMOOTHING / (V - 2)
CROW = SMOOTHING * math.log(EPS) + CONF * math.log(CONF)

_NC = 2
_NS = 16
NW = _NC * _NS
BPW = N // NW

SLABR = 64
NSLAB = 84
VSL = 4
SCW = VSL * NSLAB * SLABR

VB = 512
GRID = -(-(V - SCW) // (2 * VB))
CUT = GRID * VB
BOFF = (CUT + SCW) // VB


def _sumrows(x):
    s = x[0:8, :]
    for g in range(1, x.shape[0] // 8):
        s = s + x[g * 8:(g + 1) * 8, :]
    return s


def _tc_body(tgt_ref, xa, xb, out_ref, acc_ref, s0_ref):
    j = pl.program_id(0)
    nlast = pl.num_programs(0) - 1

    @pl.when(j == 0)
    def _init():
        m = jnp.minimum(tgt_ref[...], 1).astype(jnp.float32)
        acc_ref[...] = jnp.zeros_like(acc_ref)
        s0_ref[0, 0] = jnp.sum(xa[0:1, :] * m)

    s = _sumrows(xa[...])

    @pl.when(j != nlast)
    def _main():
        acc_ref[...] += s + _sumrows(xb[...])

    @pl.when(j == nlast)
    def _tail():
        x = xb[...]
        row = lax.broadcasted_iota(jnp.int32, x.shape, 0) + (j + BOFF) * VB
        acc_ref[...] += s + _sumrows(jnp.where(row < V, x, 0.0))
        m = jnp.minimum(tgt_ref[...], 1).astype(jnp.float32)
        a = acc_ref[...]
        srow = a[0:1, :]
        for r in range(1, 8):
            srow = srow + a[r:r + 1, :]
        out_ref[0, 0] = EPS * s0_ref[0, 0] - EPS * jnp.sum(srow * m)


@functools.cache
def _make_sc_part():
    return functools.partial(
        pl.kernel,
        out_type=jax.ShapeDtypeStruct((NW, 16), jnp.float32),
        mesh=plsc.VectorSubcoreMesh(core_axis_name="c", subcore_axis_name="s"),
        scratch_types=[
            pltpu.VMEM((BPW,), jnp.int32),
            pltpu.VMEM((128,), jnp.int32),
            pltpu.VMEM((BPW, 8, 128), jnp.float32),
            pltpu.VMEM((SLABR, 128), jnp.float32),
            pltpu.VMEM((SLABR, 128), jnp.float32),
            pltpu.VMEM((16,), jnp.float32),
            pltpu.SemaphoreType.DMA,
            pltpu.SemaphoreType.DMA,
            pltpu.SemaphoreType.DMA,
        ],
    )(_sc_body)


def _sc_body(tgt_hbm, xt_hbm, out_hbm, tgt_v, tgtc_v, blk_v, buf_a, buf_b,
             acc_v, sem, sem_a, sem_b):
    wid = lax.axis_index("s") * _NC + lax.axis_index("c")
    base = wid * BPW
    cg = wid // VSL
    vsl = wid % VSL
    colb = pl.multiple_of(cg * 128, 128)
    pltpu.sync_copy(tgt_hbm.at[pl.ds(base, BPW)], tgt_v)
    pltpu.sync_copy(tgt_hbm.at[pl.ds(colb, 128)], tgtc_v)
    t16s = [tgt_v[pl.ds(0, 16)], tgt_v[pl.ds(16, 16)]]

    copies = []
    for k in range(BPW):
        ti = t16s[k // 16][k % 16]
        rb = pl.multiple_of(jnp.bitwise_and(ti, jnp.int32(~7)), 8)
        cp = pltpu.make_async_copy(
            xt_hbm.at[pl.ds(rb, 8), pl.ds(colb, 128)], blk_v.at[k], sem)
        cp.start()
        copies.append(cp)

    row0 = CUT + vsl * (NSLAB * SLABR)

    def _start(slab_idx, buf, s):
        r = pl.multiple_of(row0 + slab_idx * SLABR, 8)
        pltpu.make_async_copy(
            xt_hbm.at[pl.ds(r, SLABR), pl.ds(colb, 128)], buf, s).start()

    def _wait(buf, s):
        pltpu.make_async_copy(
            xt_hbm.at[pl.ds(0, SLABR), pl.ds(0, 128)], buf, s).wait()

    def _accum(buf, accs):
        out = list(accs)
        for r in range(SLABR):
            for g in range(8):
                out[g] = out[g] + buf[r, pl.ds(g * 16, 16)]
        return tuple(out)

    def _loop(i, accs):
        _wait(buf_a, sem_a)
        accs = _accum(buf_a, accs)

        @pl.when(2 * i + 2 < NSLAB)
        def _():
            _start(2 * i + 2, buf_a, sem_a)

        _wait(buf_b, sem_b)
        accs = _accum(buf_b, accs)

        @pl.when(2 * i + 3 < NSLAB)
        def _():
            _start(2 * i + 3, buf_b, sem_b)

        return accs

    _start(0, buf_a, sem_a)
    _start(1, buf_b, sem_b)
    accs = lax.fori_loop(
        0, NSLAB // 2, _loop,
        tuple(jnp.zeros((16,), jnp.float32) for _ in range(8)))

    ov = jnp.zeros((16,), jnp.float32)
    for g in range(8):
        mg = jnp.minimum(tgtc_v[pl.ds(g * 16, 16)], 1).astype(jnp.float32)
        ov = ov + mg * accs[g]
    ov = (-EPS) * ov

    for cp in copies:
        cp.wait()
    lanes = lax.iota(jnp.int32, 16)
    for k in range(BPW):
        ti = t16s[k // 16][k % 16]
        tr = jnp.bitwise_and(ti, 7)
        q = pl.multiple_of((2 * vsl + k // 16) * 16, 16)
        sel = jnp.zeros((16,), jnp.float32)
        for r in range(8):
            sr = (1 - jnp.minimum(jnp.abs(tr - r), 1)).astype(jnp.float32)
            sel = sel + sr * blk_v[k, r, pl.ds(q, 16)]
        ohl = (1 - jnp.minimum(jnp.abs(lanes - (k % 16)), 1)).astype(jnp.float32)
        mf = jnp.minimum(ti, 1).astype(jnp.float32)
        ov = ov + (mf * ohl) * ((EPS - CONF) * sel + CROW)
    acc_v[...] = ov
    pltpu.sync_copy(acc_v, out_hbm.at[wid])


def kernel(x, target):
    tgt = target.astype(jnp.int32)
    xt = x.T
    sc_out = _make_sc_part()(tgt, xt)
    tc_out = pl.pallas_call(
        _tc_body,
        grid=(GRID,),
        in_specs=[
            pl.BlockSpec((1, N), lambda j: (0, 0)),
            pl.BlockSpec((VB, N), lambda j: (j, 0)),
            pl.BlockSpec((VB, N), lambda j: (j + BOFF, 0)),
        ],
        out_specs=pl.BlockSpec((1, 1), lambda j: (0, 0), memory_space=pltpu.SMEM),
        out_shape=jax.ShapeDtypeStruct((1, 1), jnp.float32),
        scratch_shapes=[
            pltpu.VMEM((8, N), jnp.float32),
            pltpu.SMEM((1, 1), jnp.float32),
        ],
    )(tgt.reshape(1, N), xt, xt)
    return tc_out[0, 0] + jnp.sum(sc_out)


# speedup vs baseline: 3.6459x; 1.2838x over previous
import functools
import math

import jax
import jax.numpy as jnp
from jax import lax
from jax.experimental import pallas as pl
from jax.experimental.pallas import tpu as pltpu
from jax.experimental.pallas import tpu_sc as plsc

V = 100000
N = 1024
PAD = 0
SMOOTHING = 0.1
CONF = 1.0 - SMOOTHING
EPS = SMOOTHING / (V - 2)
CROW = SMOOTHING * math.log(EPS) + CONF * math.log(CONF)

_NC = 2
_NS = 16
NW = _NC * _NS
BPW = N // NW

SLABR = 128
NSLAB = 28
VSL = 4
SCW = VSL * NSLAB * SLABR

VB = 512
GRID = -(-(V - SCW) // (2 * VB))
CUT = GRID * VB
BOFF = (CUT + SCW) // VB


def _sumrows(x):
    s = x[0:8, :]
    for g in range(1, x.shape[0] // 8):
        s = s + x[g * 8:(g + 1) * 8, :]
    return s


def _tc_body(tgt_ref, xa, xb, out_ref, acc_ref, s0_ref):
    j = pl.program_id(0)
    nlast = pl.num_programs(0) - 1

    @pl.when(j == 0)
    def _init():
        m = jnp.minimum(tgt_ref[...], 1).astype(jnp.float32)
        acc_ref[...] = jnp.zeros_like(acc_ref)
        s0_ref[0, 0] = jnp.sum(xa[0:1, :] * m)

    s = _sumrows(xa[...])

    @pl.when(j != nlast)
    def _main():
        acc_ref[...] += s + _sumrows(xb[...])

    @pl.when(j == nlast)
    def _tail():
        x = xb[...]
        row = lax.broadcasted_iota(jnp.int32, x.shape, 0) + (j + BOFF) * VB
        acc_ref[...] += s + _sumrows(jnp.where(row < V, x, 0.0))
        m = jnp.minimum(tgt_ref[...], 1).astype(jnp.float32)
        a = acc_ref[...]
        srow = a[0:1, :]
        for r in range(1, 8):
            srow = srow + a[r:r + 1, :]
        out_ref[0, 0] = EPS * s0_ref[0, 0] - EPS * jnp.sum(srow * m)


@functools.cache
def _make_sc_part():
    return functools.partial(
        pl.kernel,
        out_type=jax.ShapeDtypeStruct((NW, 16), jnp.float32),
        mesh=plsc.VectorSubcoreMesh(core_axis_name="c", subcore_axis_name="s"),
        scratch_types=[
            pltpu.VMEM((BPW,), jnp.int32),
            pltpu.VMEM((128,), jnp.int32),
            pltpu.VMEM((BPW, 8, 128), jnp.float32),
            pltpu.VMEM((SLABR, 128), jnp.float32),
            pltpu.VMEM((SLABR, 128), jnp.float32),
            pltpu.VMEM((16,), jnp.float32),
            pltpu.SemaphoreType.DMA,
            pltpu.SemaphoreType.DMA,
            pltpu.SemaphoreType.DMA,
        ],
    )(_sc_body)


def _sc_body(tgt_hbm, xt_hbm, out_hbm, tgt_v, tgtc_v, blk_v, buf_a, buf_b,
             acc_v, sem, sem_a, sem_b):
    wid = lax.axis_index("s") * _NC + lax.axis_index("c")
    base = wid * BPW
    cg = wid // VSL
    vsl = wid % VSL
    colb = pl.multiple_of(cg * 128, 128)
    pltpu.sync_copy(tgt_hbm.at[pl.ds(base, BPW)], tgt_v)
    pltpu.sync_copy(tgt_hbm.at[pl.ds(colb, 128)], tgtc_v)
    t16s = [tgt_v[pl.ds(0, 16)], tgt_v[pl.ds(16, 16)]]

    row0 = CUT + vsl * (NSLAB * SLABR)

    def _start(slab_idx, buf, s):
        r = pl.multiple_of(row0 + slab_idx * SLABR, 8)
        pltpu.make_async_copy(
            xt_hbm.at[pl.ds(r, SLABR), pl.ds(colb, 128)], buf, s).start()

    def _wait(buf, s):
        pltpu.make_async_copy(
            xt_hbm.at[pl.ds(0, SLABR), pl.ds(0, 128)], buf, s).wait()

    def _accum(buf, accs):
        out = list(accs)
        for r in range(SLABR):
            for g in range(8):
                out[g] = out[g] + buf[r, pl.ds(g * 16, 16)]
        return tuple(out)

    def _loop(i, accs):
        _wait(buf_a, sem_a)
        accs = _accum(buf_a, accs)

        @pl.when(2 * i + 2 < NSLAB)
        def _():
            _start(2 * i + 2, buf_a, sem_a)

        _wait(buf_b, sem_b)
        accs = _accum(buf_b, accs)

        @pl.when(2 * i + 3 < NSLAB)
        def _():
            _start(2 * i + 3, buf_b, sem_b)

        return accs

    _start(0, buf_a, sem_a)
    _start(1, buf_b, sem_b)

    copies = []
    for k in range(BPW):
        ti = t16s[k // 16][k % 16]
        rb = pl.multiple_of(jnp.bitwise_and(ti, jnp.int32(~7)), 8)
        cp = pltpu.make_async_copy(
            xt_hbm.at[pl.ds(rb, 8), pl.ds(colb, 128)], blk_v.at[k], sem)
        cp.start()
        copies.append(cp)

    accs = lax.fori_loop(
        0, NSLAB // 2, _loop,
        tuple(jnp.zeros((16,), jnp.float32) for _ in range(8)))

    ov = jnp.zeros((16,), jnp.float32)
    for g in range(8):
        mg = jnp.minimum(tgtc_v[pl.ds(g * 16, 16)], 1).astype(jnp.float32)
        ov = ov + mg * accs[g]
    ov = (-EPS) * ov

    for cp in copies:
        cp.wait()
    lanes = lax.iota(jnp.int32, 16)
    for k in range(BPW):
        ti = t16s[k // 16][k % 16]
        tr = jnp.bitwise_and(ti, 7)
        q = pl.multiple_of((2 * vsl + k // 16) * 16, 16)
        sel = jnp.zeros((16,), jnp.float32)
        for r in range(8):
            sr = (1 - jnp.minimum(jnp.abs(tr - r), 1)).astype(jnp.float32)
            sel = sel + sr * blk_v[k, r, pl.ds(q, 16)]
        ohl = (1 - jnp.minimum(jnp.abs(lanes - (k % 16)), 1)).astype(jnp.float32)
        mf = jnp.minimum(ti, 1).astype(jnp.float32)
        ov = ov + (mf * ohl) * ((EPS - CONF) * sel + CROW)
    acc_v[...] = ov
    pltpu.sync_copy(acc_v, out_hbm.at[wid])


def kernel(x, target):
    tgt = target.astype(jnp.int32)
    xt = x.T
    sc_out = _make_sc_part()(tgt, xt)
    tc_out = pl.pallas_call(
        _tc_body,
        grid=(GRID,),
        in_specs=[
            pl.BlockSpec((1, N), lambda j: (0, 0)),
            pl.BlockSpec((VB, N), lambda j: (j, 0)),
            pl.BlockSpec((VB, N), lambda j: (j + BOFF, 0)),
        ],
        out_specs=pl.BlockSpec((1, 1), lambda j: (0, 0), memory_space=pltpu.SMEM),
        out_shape=jax.ShapeDtypeStruct((1, 1), jnp.float32),
        scratch_shapes=[
            pltpu.VMEM((8, N), jnp.float32),
            pltpu.SMEM((1, 1), jnp.float32),
        ],
    )(tgt.reshape(1, N), xt, xt)
    return tc_out[0, 0] + jnp.sum(sc_out)


# speedup vs baseline: 3.9672x; 1.0881x over previous
import functools
import math

import jax
import jax.numpy as jnp
from jax import lax
from jax.experimental import pallas as pl
from jax.experimental.pallas import tpu as pltpu
from jax.experimental.pallas import tpu_sc as plsc

V = 100000
N = 1024
PAD = 0
SMOOTHING = 0.1
CONF = 1.0 - SMOOTHING
EPS = SMOOTHING / (V - 2)
CROW = SMOOTHING * math.log(EPS) + CONF * math.log(CONF)

_NC = 2
_NS = 16
NW = _NC * _NS
BPW = N // NW

SLABR = 16
NSLAB = 28
SCW = NW * NSLAB * SLABR

VB = 512
GRID = -(-(V - SCW) // (2 * VB))
CUT = GRID * VB
BOFF = (CUT + SCW) // VB


def _sumrows(x):
    s = x[0:8, :]
    for g in range(1, x.shape[0] // 8):
        s = s + x[g * 8:(g + 1) * 8, :]
    return s


def _tc_body(tgt_ref, xa, xb, out_ref, acc_ref, s0_ref):
    j = pl.program_id(0)
    nlast = pl.num_programs(0) - 1

    @pl.when(j == 0)
    def _init():
        m = jnp.minimum(tgt_ref[...], 1).astype(jnp.float32)
        acc_ref[...] = jnp.zeros_like(acc_ref)
        s0_ref[0, 0] = jnp.sum(xa[0:1, :] * m)

    s = _sumrows(xa[...])

    @pl.when(j != nlast)
    def _main():
        acc_ref[...] += s + _sumrows(xb[...])

    @pl.when(j == nlast)
    def _tail():
        x = xb[...]
        row = lax.broadcasted_iota(jnp.int32, x.shape, 0) + (j + BOFF) * VB
        acc_ref[...] += s + _sumrows(jnp.where(row < V, x, 0.0))
        m = jnp.minimum(tgt_ref[...], 1).astype(jnp.float32)
        a = acc_ref[...]
        srow = a[0:1, :]
        for r in range(1, 8):
            srow = srow + a[r:r + 1, :]
        out_ref[0, 0] = EPS * s0_ref[0, 0] - EPS * jnp.sum(srow * m)


@functools.cache
def _make_sc_part():
    return functools.partial(
        pl.kernel,
        out_type=jax.ShapeDtypeStruct((NW, 16), jnp.float32),
        mesh=plsc.VectorSubcoreMesh(core_axis_name="c", subcore_axis_name="s"),
        scratch_types=[
            pltpu.VMEM((N,), jnp.int32),
            pltpu.VMEM((BPW, 8, 128), jnp.float32),
            pltpu.VMEM((SLABR, N), jnp.float32),
            pltpu.VMEM((SLABR, N), jnp.float32),
            pltpu.VMEM((N,), jnp.float32),
            pltpu.VMEM((16,), jnp.float32),
            pltpu.SemaphoreType.DMA,
            pltpu.SemaphoreType.DMA,
            pltpu.SemaphoreType.DMA,
        ],
    )(_sc_body)


def _sc_body(tgt_hbm, xt_hbm, out_hbm, tgt_all, blk_v, buf_a, buf_b,
             accb, acc_v, sem, sem_a, sem_b):
    wid = lax.axis_index("s") * _NC + lax.axis_index("c")
    base = wid * BPW
    vq = wid % 4
    colb = pl.multiple_of((base // 128) * 128, 128)
    pltpu.sync_copy(tgt_hbm, tgt_all)
    t16s = [tgt_all[pl.ds(pl.multiple_of(base, 16), 16)],
            tgt_all[pl.ds(pl.multiple_of(base + 16, 16), 16)]]

    z = jnp.zeros((16,), jnp.float32)
    for g in range(N // 16):
        accb[pl.ds(g * 16, 16)] = z

    row0 = CUT + wid * (NSLAB * SLABR)

    def _start(slab_idx, buf, s):
        r = pl.multiple_of(row0 + slab_idx * SLABR, 8)
        pltpu.make_async_copy(xt_hbm.at[pl.ds(r, SLABR)], buf, s).start()

    def _wait(buf, s):
        pltpu.make_async_copy(xt_hbm.at[pl.ds(0, SLABR)], buf, s).wait()

    def _accum(buf):
        for g in range(N // 16):
            a = buf[0, pl.ds(g * 16, 16)]
            for r in range(1, SLABR):
                a = a + buf[r, pl.ds(g * 16, 16)]
            plsc.addupdate(accb.at[pl.ds(g * 16, 16)], a)

    def _loop(i, carry):
        _wait(buf_a, sem_a)
        _accum(buf_a)

        @pl.when(2 * i + 2 < NSLAB)
        def _():
            _start(2 * i + 2, buf_a, sem_a)

        _wait(buf_b, sem_b)
        _accum(buf_b)

        @pl.when(2 * i + 3 < NSLAB)
        def _():
            _start(2 * i + 3, buf_b, sem_b)

        return carry

    _start(0, buf_a, sem_a)
    _start(1, buf_b, sem_b)

    copies = []
    for k in range(BPW):
        ti = t16s[k // 16][k % 16]
        rb = pl.multiple_of(jnp.bitwise_and(ti, jnp.int32(~7)), 8)
        cp = pltpu.make_async_copy(
            xt_hbm.at[pl.ds(rb, 8), pl.ds(colb, 128)], blk_v.at[k], sem)
        cp.start()
        copies.append(cp)

    lax.fori_loop(0, NSLAB // 2, _loop, jnp.int32(0))

    ov = jnp.zeros((16,), jnp.float32)
    for g in range(N // 16):
        mg = jnp.minimum(tgt_all[pl.ds(g * 16, 16)], 1).astype(jnp.float32)
        ov = ov + mg * accb[pl.ds(g * 16, 16)]
    ov = (-EPS) * ov

    for cp in copies:
        cp.wait()
    lanes = lax.iota(jnp.int32, 16)
    for k in range(BPW):
        ti = t16s[k // 16][k % 16]
        tr = jnp.bitwise_and(ti, 7)
        q = pl.multiple_of((2 * vq + k // 16) * 16, 16)
        sel = jnp.zeros((16,), jnp.float32)
        for r in range(8):
            sr = (1 - jnp.minimum(jnp.abs(tr - r), 1)).astype(jnp.float32)
            sel = sel + sr * blk_v[k, r, pl.ds(q, 16)]
        ohl = (1 - jnp.minimum(jnp.abs(lanes - (k % 16)), 1)).astype(jnp.float32)
        mf = jnp.minimum(ti, 1).astype(jnp.float32)
        ov = ov + (mf * ohl) * ((EPS - CONF) * sel + CROW)
    acc_v[...] = ov
    pltpu.sync_copy(acc_v, out_hbm.at[wid])


def kernel(x, target):
    tgt = target.astype(jnp.int32)
    xt = x.T
    sc_out = _make_sc_part()(tgt, xt)
    tc_out = pl.pallas_call(
        _tc_body,
        grid=(GRID,),
        in_specs=[
            pl.BlockSpec((1, N), lambda j: (0, 0)),
            pl.BlockSpec((VB, N), lambda j: (j, 0)),
            pl.BlockSpec((VB, N), lambda j: (j + BOFF, 0)),
        ],
        out_specs=pl.BlockSpec((1, 1), lambda j: (0, 0), memory_space=pltpu.SMEM),
        out_shape=jax.ShapeDtypeStruct((1, 1), jnp.float32),
        scratch_shapes=[
            pltpu.VMEM((8, N), jnp.float32),
            pltpu.SMEM((1, 1), jnp.float32),
        ],
    )(tgt.reshape(1, N), xt, xt)
    return tc_out[0, 0] + jnp.sum(sc_out)
